# baseline (speedup 1.0000x reference)
import jax
import jax.numpy as jnp
from jax import lax
from jax.experimental import pallas as pl
from jax.experimental.pallas import tpu as pltpu


def _mlp_fused_kernel(x_ref, w1_ref, b1_ref, w2_ref, b2_ref, o_ref):
    h = lax.dot_general(
        x_ref[...], w1_ref[...],
        dimension_numbers=(((1,), (1,)), ((), ())),
        preferred_element_type=jnp.float32,
    )
    h = jnp.maximum(h + b1_ref[...], 0.0).astype(jnp.bfloat16)
    y = lax.dot_general(
        h, w2_ref[...],
        dimension_numbers=(((1,), (1,)), ((), ())),
        preferred_element_type=jnp.float32,
    )
    o_ref[...] = y + b2_ref[...]


def kernel(x, w1, b1, w2, b2):
    B, Din = x.shape
    H = w1.shape[0]
    O = w2.shape[0]

    TB = 1024
    B_pad = ((B + TB - 1) // TB) * TB
    xb = x.astype(jnp.bfloat16)
    if B_pad != B:
        xb = jnp.pad(xb, ((0, B_pad - B), (0, 0)))
    w1b = w1.astype(jnp.bfloat16)
    w2b = w2.astype(jnp.bfloat16)
    b1_2d = b1.reshape(1, H)
    b2_2d = b2.reshape(1, O)

    out = pl.pallas_call(
        _mlp_fused_kernel,
        out_shape=jax.ShapeDtypeStruct((B_pad, O), jnp.float32),
        grid=(B_pad // TB,),
        in_specs=[
            pl.BlockSpec((TB, Din), lambda i: (i, 0)),
            pl.BlockSpec((H, Din), lambda i: (0, 0)),
            pl.BlockSpec((1, H), lambda i: (0, 0)),
            pl.BlockSpec((O, H), lambda i: (0, 0)),
            pl.BlockSpec((1, O), lambda i: (0, 0)),
        ],
        out_specs=pl.BlockSpec((TB, O), lambda i: (i, 0)),
        compiler_params=pltpu.CompilerParams(
            dimension_semantics=("arbitrary",),
        ),
    )(xb, w1b, b1_2d, w2b, b2_2d)
    return out[:B] if B_pad != B else out
